# baseline (speedup 1.0000x reference)
import jax
import jax.numpy as jnp
from jax import lax
from jax.experimental import pallas as pl
from jax.experimental.pallas import tpu as pltpu
from jax.experimental.pallas import tpu_sc as plsc

B, D, HS, WS = 8, 1024, 16, 16
S = HS * WS
HEADS, K, CD = 16, 2048, 64
NTOK = HEADS * B * S

NC, NS = 2, 16
NW = NC * NS
ROWS_W = NTOK // NW
CHUNK = 128
NCHUNK = ROWS_W // CHUNK


def _argmin_body(cb_ref, x_ref, idx_ref):
    h = pl.program_id(0)
    cb = cb_ref[0]
    x = x_ref[0, 0]
    e2 = jnp.sum(cb * cb, axis=1, keepdims=True)
    x2 = jnp.sum(x * x, axis=0, keepdims=True)
    dots = jnp.dot(cb, x, preferred_element_type=jnp.float32)
    dist = (x2 - 2.0 * dots) + e2
    minv = jnp.min(dist, axis=0, keepdims=True)
    iota = lax.broadcasted_iota(jnp.int32, (K, S), 0)
    cand = jnp.where(dist == minv, iota, K)
    idx_ref[0, 0] = jnp.min(cand, axis=0) + h * K


def _compute_indices(cb, emb4):
    return pl.pallas_call(
        _argmin_body,
        grid=(HEADS, B),
        in_specs=[
            pl.BlockSpec((1, K, CD), lambda h, b: (h, 0, 0)),
            pl.BlockSpec((1, 1, CD, S), lambda h, b: (b, h, 0, 0)),
        ],
        out_specs=pl.BlockSpec((1, 1, S), lambda h, b: (h * B + b, 0, 0)),
        out_shape=jax.ShapeDtypeStruct((HEADS * B, 1, S), jnp.int32),
    )(cb, emb4)


def _gather_body(table_ref, idx_ref, out_ref, idx_v, rows_v, sem):
    wid = lax.axis_index("s") * NC + lax.axis_index("c")
    pltpu.sync_copy(idx_ref.at[pl.ds(wid * NCHUNK, NCHUNK)], idx_v)
    copies = [
        pltpu.async_copy(
            table_ref.at[idx_v.at[j]],
            rows_v.at[pl.ds(j * CHUNK, CHUNK)],
            sem,
        )
        for j in range(NCHUNK)
    ]
    for c in copies:
        c.wait()
    pltpu.sync_copy(rows_v, out_ref.at[pl.ds(wid * ROWS_W, ROWS_W)])


_gather_cache = []


def _gather(table, idx):
    if not _gather_cache:
        _gather_cache.append(pl.kernel(
            _gather_body,
            out_type=jax.ShapeDtypeStruct((NTOK, CD), jnp.float32),
            mesh=plsc.VectorSubcoreMesh(
                core_axis_name="c", subcore_axis_name="s",
                num_cores=NC, num_subcores=NS,
            ),
            scratch_types=[
                pltpu.VMEM((NCHUNK, CHUNK), jnp.int32),
                pltpu.VMEM((ROWS_W, CD), jnp.float32),
                pltpu.SemaphoreType.DMA,
            ],
            compiler_params=pltpu.CompilerParams(use_tc_tiling_on_sc=False),
        ))
    return _gather_cache[0](table, idx)


def kernel(embeddings, codebook):
    emb4 = embeddings.reshape(B, HEADS, CD, S)
    idx = _compute_indices(codebook, emb4)
    rows = _gather(codebook.reshape(HEADS * K, CD),
                   idx.reshape(NW * NCHUNK, CHUNK))
    q = rows.reshape(HEADS, B, S, CD).transpose(1, 0, 3, 2)
    return q.reshape(B, D, HS, WS)


# speedup vs baseline: 1.2418x; 1.2418x over previous
import jax
import jax.numpy as jnp
from jax import lax
from jax.experimental import pallas as pl
from jax.experimental.pallas import tpu as pltpu
from jax.experimental.pallas import tpu_sc as plsc

B, D, HS, WS = 8, 1024, 16, 16
S = HS * WS
HEADS, K, CD = 16, 2048, 64
NTOK = HEADS * B * S

NC, NS = 2, 16
NW = NC * NS
ROWS_W = NTOK // NW
CHUNK = 128
NCHUNK = ROWS_W // CHUNK


def _argmin_body(cb_ref, x_ref, idx_ref):
    h = pl.program_id(0)
    cb = cb_ref[0]
    e2 = jnp.sum(cb * cb, axis=1, keepdims=True)
    cb2 = cb + cb
    iota = lax.broadcasted_iota(jnp.int32, (K, S), 0).astype(jnp.float32)
    for b in range(B):
        x = x_ref[b, 0]
        x2 = jnp.sum(x * x, axis=0, keepdims=True)
        dots2 = jnp.dot(cb2, x, preferred_element_type=jnp.float32)
        dist = (x2 - dots2) + e2
        minv = jnp.min(dist, axis=0, keepdims=True)
        cand = jnp.where(dist == minv, iota, jnp.float32(K))
        idx = jnp.min(cand, axis=0).astype(jnp.int32)
        idx_ref[b, 0] = idx + h * K


def _compute_indices(cb, emb4):
    return pl.pallas_call(
        _argmin_body,
        grid=(HEADS,),
        in_specs=[
            pl.BlockSpec((1, K, CD), lambda h: (h, 0, 0)),
            pl.BlockSpec((B, 1, CD, S), lambda h: (0, h, 0, 0)),
        ],
        out_specs=pl.BlockSpec((B, 1, S), lambda h: (h, 0, 0)),
        out_shape=jax.ShapeDtypeStruct((HEADS * B, 1, S), jnp.int32),
    )(cb, emb4)


def _gather_body(table_ref, idx_ref, out_ref, idx_v, rows_v, sem):
    wid = lax.axis_index("s") * NC + lax.axis_index("c")
    pltpu.sync_copy(idx_ref.at[pl.ds(wid * NCHUNK, NCHUNK)], idx_v)
    copies = [
        pltpu.async_copy(
            table_ref.at[idx_v.at[j]],
            rows_v.at[pl.ds(j * CHUNK, CHUNK)],
            sem,
        )
        for j in range(NCHUNK)
    ]
    for c in copies:
        c.wait()
    pltpu.sync_copy(rows_v, out_ref.at[pl.ds(wid * ROWS_W, ROWS_W)])


_gather_cache = []


def _gather(table, idx):
    if not _gather_cache:
        _gather_cache.append(pl.kernel(
            _gather_body,
            out_type=jax.ShapeDtypeStruct((NTOK, CD), jnp.float32),
            mesh=plsc.VectorSubcoreMesh(
                core_axis_name="c", subcore_axis_name="s",
                num_cores=NC, num_subcores=NS,
            ),
            scratch_types=[
                pltpu.VMEM((NCHUNK, CHUNK), jnp.int32),
                pltpu.VMEM((ROWS_W, CD), jnp.float32),
                pltpu.SemaphoreType.DMA,
            ],
            compiler_params=pltpu.CompilerParams(use_tc_tiling_on_sc=False),
        ))
    return _gather_cache[0](table, idx)


def kernel(embeddings, codebook):
    emb4 = embeddings.reshape(B, HEADS, CD, S)
    idx = _compute_indices(codebook, emb4)
    rows = _gather(codebook.reshape(HEADS * K, CD),
                   idx.reshape(NW * NCHUNK, CHUNK))
    q = rows.reshape(HEADS, B, S, CD).transpose(1, 0, 3, 2)
    return q.reshape(B, D, HS, WS)


# speedup vs baseline: 1.3685x; 1.1020x over previous
import jax
import jax.numpy as jnp
from jax import lax
from jax.experimental import pallas as pl
from jax.experimental.pallas import tpu as pltpu
from jax.experimental.pallas import tpu_sc as plsc

B, D, HS, WS = 8, 1024, 16, 16
S = HS * WS
HEADS, K, CD = 16, 2048, 64
NTOK = HEADS * B * S

NC, NS = 2, 16
NW = NC * NS
ROWS_W = NTOK // NW
CHUNK = 128
NCHUNK = ROWS_W // CHUNK


def _argmin_body(cb_ref, x_ref, idx_ref):
    h = pl.program_id(0)
    cb = cb_ref[0]
    e2 = jnp.sum(cb * cb, axis=1, keepdims=True)
    cb2 = cb + cb
    iota = lax.broadcasted_iota(jnp.int32, (K, S), 0).astype(jnp.float32)
    for b in range(B):
        x = x_ref[b]
        x2 = jnp.sum(x * x, axis=0, keepdims=True)
        dots2 = jnp.dot(cb2, x, preferred_element_type=jnp.float32)
        dist = (x2 - dots2) + e2
        minv = jnp.min(dist, axis=0, keepdims=True)
        cand = jnp.where(dist == minv, iota, jnp.float32(K))
        idx = jnp.min(cand, axis=0).astype(jnp.int32)
        idx_ref[b, 0] = idx + h * K


def _compute_indices(cb, emb3):
    return pl.pallas_call(
        _argmin_body,
        grid=(HEADS,),
        in_specs=[
            pl.BlockSpec((1, K, CD), lambda h: (h, 0, 0)),
            pl.BlockSpec((B, CD, S), lambda h: (0, h, 0)),
        ],
        out_specs=pl.BlockSpec((B, 1, S), lambda h: (h, 0, 0)),
        out_shape=jax.ShapeDtypeStruct((HEADS * B, 1, S), jnp.int32),
    )(cb, emb3)


BH_W = ROWS_W // S


def _gather_body(table_ref, idx_ref, out_ref, idx_v, rows_v, sem):
    wid = lax.axis_index("s") * NC + lax.axis_index("c")
    pltpu.sync_copy(idx_ref.at[pl.ds(wid * NCHUNK, NCHUNK)], idx_v)
    copies = [
        pltpu.async_copy(
            table_ref.at[idx_v.at[j]],
            rows_v.at[j // 2, pl.ds((j % 2) * CHUNK, CHUNK)],
            sem,
        )
        for j in range(NCHUNK)
    ]
    for c in copies:
        c.wait()
    pltpu.sync_copy(rows_v, out_ref.at[pl.ds(wid * BH_W, BH_W)])


_gather_cache = []


def _gather(table, idx):
    if not _gather_cache:
        _gather_cache.append(pl.kernel(
            _gather_body,
            out_type=jax.ShapeDtypeStruct((HEADS * B, S, CD), jnp.float32),
            mesh=plsc.VectorSubcoreMesh(
                core_axis_name="c", subcore_axis_name="s",
                num_cores=NC, num_subcores=NS,
            ),
            scratch_types=[
                pltpu.VMEM((NCHUNK, CHUNK), jnp.int32),
                pltpu.VMEM((BH_W, S, CD), jnp.float32),
                pltpu.SemaphoreType.DMA,
            ],
            compiler_params=pltpu.CompilerParams(use_tc_tiling_on_sc=False),
        ))
    return _gather_cache[0](table, idx)


def kernel(embeddings, codebook):
    emb3 = embeddings.reshape(B, D, S)
    idx = _compute_indices(codebook, emb3)
    rows = _gather(codebook.reshape(HEADS * K, CD),
                   idx.reshape(NW * NCHUNK, CHUNK))
    q = rows.reshape(HEADS, B, S, CD).transpose(1, 0, 3, 2)
    return q.reshape(B, D, HS, WS)


# speedup vs baseline: 1.6074x; 1.1746x over previous
import jax
import jax.numpy as jnp
from jax import lax
from jax.experimental import pallas as pl
from jax.experimental.pallas import tpu as pltpu
from jax.experimental.pallas import tpu_sc as plsc

B, D, HS, WS = 8, 1024, 16, 16
S = HS * WS
HEADS, K, CD = 16, 2048, 64
NTOK = HEADS * B * S

NC, NS = 2, 16
NW = NC * NS
ROWS_W = NTOK // NW
CHUNK = 128
NCHUNK = ROWS_W // CHUNK


def _argmin_body(cb_ref, x_ref, idx_ref):
    h = pl.program_id(0)
    cb = cb_ref[0]
    e2 = jnp.sum(cb * cb, axis=1, keepdims=True)
    cb2 = cb + cb
    for b in range(B):
        x = x_ref[b]
        x2 = jnp.sum(x * x, axis=0, keepdims=True)
        dots2 = jnp.dot(cb2, x, preferred_element_type=jnp.float32)
        dist = (x2 - dots2) + e2
        idx = jnp.argmin(dist, axis=0).astype(jnp.int32)
        idx_ref[b, 0] = idx + h * K


def _compute_indices(cb, emb3):
    return pl.pallas_call(
        _argmin_body,
        grid=(HEADS,),
        in_specs=[
            pl.BlockSpec((1, K, CD), lambda h: (h, 0, 0)),
            pl.BlockSpec((B, CD, S), lambda h: (0, h, 0)),
        ],
        out_specs=pl.BlockSpec((B, 1, S), lambda h: (h, 0, 0)),
        out_shape=jax.ShapeDtypeStruct((HEADS * B, 1, S), jnp.int32),
    )(cb, emb3)


BH_W = ROWS_W // S


def _gather_body(table_ref, idx_ref, out_ref, idx_v, rows_v, sem):
    wid = lax.axis_index("s") * NC + lax.axis_index("c")
    pltpu.sync_copy(idx_ref.at[pl.ds(wid * NCHUNK, NCHUNK)], idx_v)
    copies = [
        pltpu.async_copy(
            table_ref.at[idx_v.at[j]],
            rows_v.at[j // 2, pl.ds((j % 2) * CHUNK, CHUNK)],
            sem,
        )
        for j in range(NCHUNK)
    ]
    for c in copies:
        c.wait()
    pltpu.sync_copy(rows_v, out_ref.at[pl.ds(wid * BH_W, BH_W)])


_gather_cache = []


def _gather(table, idx):
    if not _gather_cache:
        _gather_cache.append(pl.kernel(
            _gather_body,
            out_type=jax.ShapeDtypeStruct((HEADS * B, S, CD), jnp.float32),
            mesh=plsc.VectorSubcoreMesh(
                core_axis_name="c", subcore_axis_name="s",
                num_cores=NC, num_subcores=NS,
            ),
            scratch_types=[
                pltpu.VMEM((NCHUNK, CHUNK), jnp.int32),
                pltpu.VMEM((BH_W, S, CD), jnp.float32),
                pltpu.SemaphoreType.DMA,
            ],
            compiler_params=pltpu.CompilerParams(use_tc_tiling_on_sc=False),
        ))
    return _gather_cache[0](table, idx)


def kernel(embeddings, codebook):
    emb3 = embeddings.reshape(B, D, S)
    idx = _compute_indices(codebook, emb3)
    rows = _gather(codebook.reshape(HEADS * K, CD),
                   idx.reshape(NW * NCHUNK, CHUNK))
    q = rows.reshape(HEADS, B, S, CD).transpose(1, 0, 3, 2)
    return q.reshape(B, D, HS, WS)
